# baseline (speedup 1.0000x reference)
import functools

import jax
import jax.numpy as jnp
from jax import lax
from jax.experimental import pallas as pl
from jax.experimental.pallas import tpu as pltpu
from jax.experimental.pallas import tpu_sc as plsc

_L = 200
_D = 64
_C = 64
_LANES = 16
_LP = 208
_NCHUNK = _LP // _LANES
_RP_W = 2 * _L + _LANES

_NC = 2
_NS = 16
_NW = _NC * _NS


def _sc_body(rows_per_w, x_hbm, slen_hbm, fp_hbm, rp_hbm, out_hbm,
             slen_v, fp_v, rp_v, buf_v):
    cid = lax.axis_index("c")
    sid = lax.axis_index("s")
    wid = sid * _NC + cid
    base = wid * rows_per_w

    pltpu.sync_copy(slen_hbm.at[pl.ds(base, rows_per_w)],
                    slen_v.at[pl.ds(0, rows_per_w)])
    pltpu.sync_copy(fp_hbm, fp_v)
    pltpu.sync_copy(rp_hbm, rp_v)

    lanes = lax.iota(jnp.int32, 16)

    def row_body(i, carry):
        b = base + i
        s = slen_v[pl.ds(i, _LANES)][0]
        pltpu.sync_copy(x_hbm.at[b], out_hbm.at[b, pl.ds(0, _C), :])

        start = _L - s

        def d_body(d, carry2):
            for c in range(_NCHUNK):
                cb = c * _LANES
                col = lanes + cb
                fv = fp_v[d, pl.ds(cb, _LANES)]
                fv = jnp.where(col < s, fv, 0.0)
                buf_v[d, pl.ds(cb, _LANES)] = fv
                rv = rp_v[d, pl.ds(start + cb, _LANES)]
                buf_v[_D + d, pl.ds(cb, _LANES)] = rv
            return carry2

        lax.fori_loop(0, _D, d_body, 0)
        pltpu.sync_copy(buf_v.at[:, pl.ds(0, _L)],
                        out_hbm.at[b, pl.ds(_C, 2 * _D), :])
        return carry

    lax.fori_loop(0, rows_per_w, row_body, 0)


def kernel(x, seq_len, W):
    B, C, L = x.shape
    assert (C, L) == (_C, _L) and W.shape == (_L + 1, _D)
    assert B % _NW == 0
    rows_per_w = B // _NW

    F = W[1:_L + 1, :].T.astype(jnp.float32)
    Fp = jnp.pad(F, ((0, 0), (0, _LP - _L)))
    Rp = jnp.pad(F[:, ::-1], ((0, 0), (0, _RP_W - _L)))

    mesh = plsc.VectorSubcoreMesh(core_axis_name="c", subcore_axis_name="s",
                                  num_cores=_NC, num_subcores=_NS)
    run = pl.kernel(
        functools.partial(_sc_body, rows_per_w),
        out_type=jax.ShapeDtypeStruct((B, _C + 2 * _D, _L), jnp.float32),
        mesh=mesh,
        compiler_params=pltpu.CompilerParams(use_tc_tiling_on_sc=False),
        scratch_types=[
            pltpu.VMEM((rows_per_w + _LANES,), jnp.int32),
            pltpu.VMEM((_D, _LP), jnp.float32),
            pltpu.VMEM((_D, _RP_W), jnp.float32),
            pltpu.VMEM((2 * _D, _LP), jnp.float32),
        ],
    )
    return run(x, seq_len.astype(jnp.int32), Fp, Rp)


# speedup vs baseline: 1.0014x; 1.0014x over previous
import functools

import jax
import jax.numpy as jnp
from jax import lax
from jax.experimental import pallas as pl
from jax.experimental.pallas import tpu as pltpu
from jax.experimental.pallas import tpu_sc as plsc

_L = 200
_D = 64
_C = 64
_LANES = 16
_LP = 208
_NCHUNK = _LP // _LANES
_RP_W = 2 * _L + _LANES

_NC = 2
_NS = 16
_NW = _NC * _NS


def _sc_body(rows_per_w, x_hbm, slen_hbm, fp_hbm, rp_hbm, out_hbm,
             slen_v, fp_v, rp_v, buf_v, xsem, esem):
    cid = lax.axis_index("c")
    sid = lax.axis_index("s")
    wid = sid * _NC + cid
    base = wid * rows_per_w

    xcopy = pltpu.async_copy(
        x_hbm.at[pl.ds(base, rows_per_w)],
        out_hbm.at[pl.ds(base, rows_per_w), pl.ds(0, _C), :],
        xsem)

    pltpu.sync_copy(slen_hbm.at[pl.ds(base, rows_per_w)],
                    slen_v.at[pl.ds(0, rows_per_w)])
    pltpu.sync_copy(fp_hbm, fp_v)
    pltpu.sync_copy(rp_hbm, rp_v)

    lanes = lax.iota(jnp.int32, 16)

    def emb_copy(p, b, sem_slot):
        return pltpu.make_async_copy(
            buf_v.at[p, :, pl.ds(0, _L)],
            out_hbm.at[b, pl.ds(_C, 2 * _D), :],
            esem.at[sem_slot])

    def row_body(i, carry):
        b = base + i
        p = lax.rem(i, 2)
        s = slen_v[pl.ds(i, _LANES)][0]
        start = _L - s

        @pl.when(i >= 2)
        def _():
            emb_copy(p, b, p).wait()

        def d_body(d, carry2):
            for c in range(_NCHUNK):
                cb = c * _LANES
                col = lanes + cb
                fv = fp_v[d, pl.ds(cb, _LANES)]
                fv = jnp.where(col < s, fv, 0.0)
                buf_v[p, d, pl.ds(cb, _LANES)] = fv
                rv = rp_v[d, pl.ds(start + cb, _LANES)]
                buf_v[p, _D + d, pl.ds(cb, _LANES)] = rv
            return carry2

        lax.fori_loop(0, _D, d_body, 0)
        emb_copy(p, b, p).start()
        return carry

    lax.fori_loop(0, rows_per_w, row_body, 0)
    emb_copy(0, base, 0).wait()
    emb_copy(1, base, 1).wait()
    xcopy.wait()


def kernel(x, seq_len, W):
    B, C, L = x.shape
    assert (C, L) == (_C, _L) and W.shape == (_L + 1, _D)
    assert B % _NW == 0
    rows_per_w = B // _NW

    F = W[1:_L + 1, :].T.astype(jnp.float32)
    Fp = jnp.pad(F, ((0, 0), (0, _LP - _L)))
    Rp = jnp.pad(F[:, ::-1], ((0, 0), (0, _RP_W - _L)))

    mesh = plsc.VectorSubcoreMesh(core_axis_name="c", subcore_axis_name="s",
                                  num_cores=_NC, num_subcores=_NS)
    run = pl.kernel(
        functools.partial(_sc_body, rows_per_w),
        out_type=jax.ShapeDtypeStruct((B, _C + 2 * _D, _L), jnp.float32),
        mesh=mesh,
        compiler_params=pltpu.CompilerParams(use_tc_tiling_on_sc=False),
        scratch_types=[
            pltpu.VMEM((rows_per_w + _LANES,), jnp.int32),
            pltpu.VMEM((_D, _LP), jnp.float32),
            pltpu.VMEM((_D, _RP_W), jnp.float32),
            pltpu.VMEM((2, 2 * _D, _LP), jnp.float32),
            pltpu.SemaphoreType.DMA,
            pltpu.SemaphoreType.DMA((2,)),
        ],
    )
    return run(x, seq_len.astype(jnp.int32), Fp, Rp)
